# baseline (speedup 1.0000x reference)
import functools

import jax
import jax.numpy as jnp
from jax import lax
from jax.experimental import pallas as pl
from jax.experimental.pallas import tpu as pltpu
from jax.experimental.pallas import tpu_sc as plsc

_E = 3_200_000
_N = 100_000
_RSR_PAD = 1024
_RTG_PAD = 1280

_NC = 2
_NS = 16
_NW = _NC * _NS
_L = 16

_NPAD = 100_352
_SLICE = _NPAD // _NS
_EPW = _E // _NW
_C = 2000
_NCHUNK = _EPW // _C
_IW = 80
_IR = _C // _IW
_ERW = _EPW // _IW
_ND = _NPAD // _NW
_NLAST = _N - (_NW - 1) * _ND


def _relw_body(a_ref, b_ref, wsr_ref, wtg_ref):
    a = a_ref[...]
    b = b_ref[...]
    pa = a / (jnp.sqrt(jnp.sum(a * a, axis=1, keepdims=True)) + 1e-8)
    pb = b / (jnp.sqrt(jnp.sum(b * b, axis=1, keepdims=True)) + 1e-8)
    sim = lax.dot_general(pa, pb, (((1,), (1,)), ((), ())),
                          preferred_element_type=jnp.float32)
    ii = lax.broadcasted_iota(jnp.int32, sim.shape, 0)
    jj = lax.broadcasted_iota(jnp.int32, sim.shape, 1)
    neg = jnp.float32(-3.0e38)
    wsr_ref[...] = jnp.max(jnp.where(jj < 1200, sim, neg), axis=1, keepdims=True)
    wtg_ref[...] = jnp.max(jnp.where(ii < 1200, sim, neg), axis=0, keepdims=True)


def _relation_w(a, b):
    a = jnp.pad(a, ((0, 1280 - a.shape[0]), (0, 0)))
    b = jnp.pad(b, ((0, 1280 - b.shape[0]), (0, 0)))
    wsr2, wtg2 = pl.pallas_call(
        _relw_body,
        out_shape=(jax.ShapeDtypeStruct((1280, 1), jnp.float32),
                   jax.ShapeDtypeStruct((1, 1280), jnp.float32)),
    )(a, b)
    return wsr2[:_RSR_PAD, 0], wtg2[0, :]


def _dis_body(dsr_ref, dtg_ref, dis_sr_ref, d2_sr_ref, dis_tg_ref, d2_tg_ref):
    for dref, oref, o2ref in ((dsr_ref, dis_sr_ref, d2_sr_ref),
                              (dtg_ref, dis_tg_ref, d2_tg_ref)):
        d = dref[0] + dref[1] + 1.0
        y = jnp.where(d > 0.0, lax.rsqrt(jnp.maximum(d, 1e-12)), 0.0)
        oref[...] = y
        o2ref[...] = y * y


def _deg_to_dis(deg_sr, deg_tg):
    shp = jax.ShapeDtypeStruct((_NPAD // 128, 128), jnp.float32)
    return pl.pallas_call(
        _dis_body,
        out_shape=(shp, shp, shp, shp),
    )(deg_sr.reshape(2, _NPAD // 128, 128), deg_tg.reshape(2, _NPAD // 128, 128))


def _make_pass1():
    mesh = plsc.VectorSubcoreMesh(core_axis_name="c", subcore_axis_name="s")

    @functools.partial(
        pl.kernel,
        mesh=mesh,
        compiler_params=pltpu.CompilerParams(needs_layout_passes=False,
                                             use_tc_tiling_on_sc=False),
        out_type=[jax.ShapeDtypeStruct((_E,), jnp.float32),
                  jax.ShapeDtypeStruct((_E,), jnp.float32),
                  jax.ShapeDtypeStruct((2 * _NPAD,), jnp.float32),
                  jax.ShapeDtypeStruct((2 * _NPAD,), jnp.float32)],
        scratch_types=[
            pltpu.VMEM((_RSR_PAD,), jnp.float32),
            pltpu.VMEM((_RTG_PAD,), jnp.float32),
            [pltpu.VMEM((_C,), jnp.int32) for _ in range(2)],
            [pltpu.VMEM((_C,), jnp.float32) for _ in range(2)],
            [pltpu.VMEM((_C,), jnp.float32) for _ in range(2)],
            [pltpu.VMEM((_C,), jnp.float32) for _ in range(2)],
            [pltpu.VMEM((_IR, _IW), jnp.int32) for _ in range(2)],
            [pltpu.VMEM((_C,), jnp.float32) for _ in range(2)],
            pltpu.VMEM_SHARED((2 * _NPAD,), jnp.float32),
            [pltpu.SemaphoreType.DMA for _ in range(2)],
            [pltpu.SemaphoreType.DMA for _ in range(2)],
            [pltpu.SemaphoreType.DMA for _ in range(2)],
            [pltpu.SemaphoreType.DMA for _ in range(2)],
        ],
    )
    def pass1(wsr_hbm, wtg_hbm,
              rel_sr, conf_sr, imp_sr, pca_sr, rows_sr,
              rel_tg, conf_tg, imp_tg, pca_tg, rows_tg,
              zeros_hbm,
              vals_sr, vals_tg, deg_sr, deg_tg,
              wsr_v, wtg_v, rel_v, conf_v, imp_v, pca_v, rows_v, vals_v,
              deg_sh, dsem, rsem, osem, ssem):
        cid = lax.axis_index("c")
        sid = lax.axis_index("s")
        wid = sid * _NC + cid
        pltpu.sync_copy(zeros_hbm.at[pl.ds(sid * 2 * _SLICE, 2 * _SLICE)],
                        deg_sh.at[pl.ds(sid * 2 * _SLICE, 2 * _SLICE)])
        pltpu.sync_copy(wsr_hbm, wsr_v)
        pltpu.sync_copy(wtg_hbm, wtg_v)
        plsc.subcore_barrier()

        for w_v, rel_hbm, conf_hbm, imp_hbm, pca_hbm, rows2_hbm, vals_hbm, \
                doff in ((wsr_v, rel_sr, conf_sr, imp_sr, pca_sr, rows_sr,
                          vals_sr, 0),
                         (wtg_v, rel_tg, conf_tg, imp_tg, pca_tg, rows_tg,
                          vals_tg, _NPAD)):
            deg_side = deg_sh.at[pl.ds(doff, _NPAD)]

            def stage4(c, b):
                base = wid * _EPW + c * _C
                return (pltpu.make_async_copy(rel_hbm.at[pl.ds(base, _C)], rel_v[b], dsem[b]),
                        pltpu.make_async_copy(conf_hbm.at[pl.ds(base, _C)], conf_v[b], dsem[b]),
                        pltpu.make_async_copy(imp_hbm.at[pl.ds(base, _C)], imp_v[b], dsem[b]),
                        pltpu.make_async_copy(pca_hbm.at[pl.ds(base, _C)], pca_v[b], dsem[b]))

            def rows_cp(c, b):
                rbase = wid * _ERW + c * _IR
                return pltpu.make_async_copy(rows2_hbm.at[pl.ds(rbase, _IR)],
                                             rows_v[b], rsem[b])

            def wb_cp(c, b):
                base = wid * _EPW + c * _C
                return pltpu.make_async_copy(vals_v[b],
                                             vals_hbm.at[pl.ds(base, _C)],
                                             osem[b])

            def scat_cps(b):
                return [pltpu.make_async_copy(
                            vals_v[b].at[pl.ds(j * _IW, _IW)],
                            deg_side.at[rows_v[b].at[j]], ssem[b])
                        for j in range(_IR)]

            for b in range(2):
                for cp in stage4(b, b):
                    cp.start()
                rows_cp(b, b).start()

            @pl.loop(0, _NCHUNK, step=2)
            def _(g):
                for b in range(2):
                    c = g + b

                    @pl.when(c >= 2)
                    def _():
                        for cp in scat_cps(b):
                            cp.wait()
                        wb_cp(c - 2, b).wait()
                        rows_cp(c, b).start()

                    for cp in stage4(c, b):
                        cp.wait()

                    @plsc.parallel_loop(0, _C, _L, unroll=25)
                    def _(o):
                        att = plsc.load_gather(w_v, [rel_v[b][pl.ds(o, _L)]])
                        v = (conf_v[b][pl.ds(o, _L)] * imp_v[b][pl.ds(o, _L)]
                             * (0.5 * pca_v[b][pl.ds(o, _L)] + 0.5 * att))
                        vals_v[b][pl.ds(o, _L)] = v

                    rows_cp(c, b).wait()
                    wb_cp(c, b).start()
                    for cp in scat_cps(b):
                        cp.start(add=True)

                    @pl.when(c + 2 < _NCHUNK)
                    def _():
                        for cp in stage4(c + 2, b):
                            cp.start()

            for b in range(2):
                for cp in scat_cps(b):
                    cp.wait()
                wb_cp(_NCHUNK - 2 + b, b).wait()

        plsc.subcore_barrier()
        pltpu.sync_copy(deg_sh.at[pl.ds(sid * _SLICE, _SLICE)],
                        deg_sr.at[pl.ds(cid * _NPAD + sid * _SLICE, _SLICE)])
        pltpu.sync_copy(deg_sh.at[pl.ds(_NPAD + sid * _SLICE, _SLICE)],
                        deg_tg.at[pl.ds(cid * _NPAD + sid * _SLICE, _SLICE)])

    return pass1


def _make_pass2():
    mesh = plsc.VectorSubcoreMesh(core_axis_name="c", subcore_axis_name="s")

    @functools.partial(
        pl.kernel,
        mesh=mesh,
        compiler_params=pltpu.CompilerParams(needs_layout_passes=False,
                                             use_tc_tiling_on_sc=False),
        out_type=[jax.ShapeDtypeStruct((_E + _N,), jnp.float32),
                  jax.ShapeDtypeStruct((_E + _N,), jnp.float32)],
        scratch_types=[
            [pltpu.VMEM((_C,), jnp.int32) for _ in range(2)],
            [pltpu.VMEM((_C,), jnp.int32) for _ in range(2)],
            [pltpu.VMEM((_C,), jnp.float32) for _ in range(2)],
            [pltpu.VMEM((_C,), jnp.float32) for _ in range(2)],
            pltpu.VMEM((_NPAD,), jnp.float32),
            pltpu.VMEM((_ND,), jnp.float32),
            [pltpu.SemaphoreType.DMA for _ in range(2)],
            [pltpu.SemaphoreType.DMA for _ in range(2)],
        ],
    )
    def pass2(dis_sr, d2_sr, rows_sr, cols_sr, vraw_sr,
              dis_tg, d2_tg, rows_tg, cols_tg, vraw_tg,
              vout_sr, vout_tg,
              rows_v, cols_v, vals_v, out_v, dis_full, diag_v, dsem, osem):
        cid = lax.axis_index("c")
        sid = lax.axis_index("s")
        wid = sid * _NC + cid

        for dis_hbm, d2_hbm, rows_hbm, cols_hbm, vraw_hbm, vout_hbm in (
                (dis_sr, d2_sr, rows_sr, cols_sr, vraw_sr, vout_sr),
                (dis_tg, d2_tg, rows_tg, cols_tg, vraw_tg, vout_tg)):
            pltpu.sync_copy(dis_hbm, dis_full)
            pltpu.sync_copy(d2_hbm.at[pl.ds(wid * _ND, _ND)], diag_v)

            @pl.when(wid < _NW - 1)
            def _():
                pltpu.sync_copy(diag_v,
                                vout_hbm.at[pl.ds(_E + wid * _ND, _ND)])

            @pl.when(wid == _NW - 1)
            def _():
                pltpu.sync_copy(diag_v.at[pl.ds(0, _NLAST)],
                                vout_hbm.at[pl.ds(_E + (_NW - 1) * _ND,
                                                  _NLAST)])

            def stage3(c, b):
                base = wid * _EPW + c * _C
                return (pltpu.make_async_copy(rows_hbm.at[pl.ds(base, _C)], rows_v[b], dsem[b]),
                        pltpu.make_async_copy(cols_hbm.at[pl.ds(base, _C)], cols_v[b], dsem[b]),
                        pltpu.make_async_copy(vraw_hbm.at[pl.ds(base, _C)], vals_v[b], dsem[b]))

            def wb_cp(c, b):
                base = wid * _EPW + c * _C
                return pltpu.make_async_copy(out_v[b],
                                             vout_hbm.at[pl.ds(base, _C)],
                                             osem[b])

            for b in range(2):
                for cp in stage3(b, b):
                    cp.start()

            @pl.loop(0, _NCHUNK, step=2)
            def _(g):
                for b in range(2):
                    c = g + b

                    @pl.when(c >= 2)
                    def _():
                        wb_cp(c - 2, b).wait()

                    for cp in stage3(c, b):
                        cp.wait()

                    @plsc.parallel_loop(0, _C, _L, unroll=25)
                    def _(o):
                        dr = plsc.load_gather(dis_full, [rows_v[b][pl.ds(o, _L)]])
                        dc = plsc.load_gather(dis_full, [cols_v[b][pl.ds(o, _L)]])
                        out_v[b][pl.ds(o, _L)] = vals_v[b][pl.ds(o, _L)] * dr * dc

                    wb_cp(c, b).start()

                    @pl.when(c + 2 < _NCHUNK)
                    def _():
                        for cp in stage3(c + 2, b):
                            cp.start()

            for b in range(2):
                wb_cp(_NCHUNK - 2 + b, b).wait()

    return pass2


_pass1 = _make_pass1()
_pass2 = _make_pass2()


def kernel(rel_sr_weight, rel_tg_weight, pos_sr, relation_sr, conf_sr,
           imp_sr, pca_sr, pos_tg, relation_tg, conf_tg, imp_tg, pca_tg):
    w_sr, w_tg = _relation_w(rel_sr_weight, rel_tg_weight)
    zeros = jnp.zeros((2 * _NPAD,), jnp.float32)

    vraw_sr, vraw_tg, deg_sr, deg_tg = _pass1(
        w_sr, w_tg,
        relation_sr, conf_sr, imp_sr, pca_sr,
        pos_sr[0].reshape(_E // _IW, _IW),
        relation_tg, conf_tg, imp_tg, pca_tg,
        pos_tg[0].reshape(_E // _IW, _IW),
        zeros)
    dis_sr, d2_sr, dis_tg, d2_tg = _deg_to_dis(deg_sr, deg_tg)
    diag = jnp.arange(_N, dtype=jnp.int32)
    vout_sr, vout_tg = _pass2(
        dis_sr.reshape(-1), d2_sr.reshape(-1), pos_sr[0], pos_sr[1], vraw_sr,
        dis_tg.reshape(-1), d2_tg.reshape(-1), pos_tg[0], pos_tg[1], vraw_tg)

    out = []
    for pos, vout in ((pos_sr, vout_sr), (pos_tg, vout_tg)):
        rows = jnp.concatenate([pos[0], diag])
        cols = jnp.concatenate([pos[1], diag])
        out.extend([rows, cols, vout])
    return tuple(out)


# speedup vs baseline: 1.0075x; 1.0075x over previous
import functools

import jax
import jax.numpy as jnp
from jax import lax
from jax.experimental import pallas as pl
from jax.experimental.pallas import tpu as pltpu
from jax.experimental.pallas import tpu_sc as plsc

_E = 3_200_000
_N = 100_000
_RSR_PAD = 1024
_RTG_PAD = 1280

_NC = 2
_NS = 16
_NW = _NC * _NS
_L = 16

_NPAD = 100_352
_SLICE = _NPAD // _NS
_EPW = _E // _NW
_C = 2000
_NCHUNK = _EPW // _C
_IW = 80
_IR = _C // _IW
_ERW = _EPW // _IW
_ND = _NPAD // _NW
_NLAST = _N - (_NW - 1) * _ND


def _relw_body(a_ref, b_ref, wsr_ref, wtg_ref):
    a = a_ref[...]
    b = b_ref[...]
    pa = a / (jnp.sqrt(jnp.sum(a * a, axis=1, keepdims=True)) + 1e-8)
    pb = b / (jnp.sqrt(jnp.sum(b * b, axis=1, keepdims=True)) + 1e-8)
    sim = lax.dot_general(pa, pb, (((1,), (1,)), ((), ())),
                          preferred_element_type=jnp.float32)
    ii = lax.broadcasted_iota(jnp.int32, sim.shape, 0)
    jj = lax.broadcasted_iota(jnp.int32, sim.shape, 1)
    neg = jnp.float32(-3.0e38)
    wsr_ref[...] = jnp.max(jnp.where(jj < 1200, sim, neg), axis=1, keepdims=True)
    wtg_ref[...] = jnp.max(jnp.where(ii < 1200, sim, neg), axis=0, keepdims=True)


def _relation_w(a, b):
    a = jnp.pad(a, ((0, 1280 - a.shape[0]), (0, 0)))
    b = jnp.pad(b, ((0, 1280 - b.shape[0]), (0, 0)))
    wsr2, wtg2 = pl.pallas_call(
        _relw_body,
        out_shape=(jax.ShapeDtypeStruct((1280, 1), jnp.float32),
                   jax.ShapeDtypeStruct((1, 1280), jnp.float32)),
    )(a, b)
    return wsr2[:_RSR_PAD, 0], wtg2[0, :]


def _rsqrt_vec(x):
    i = plsc.bitcast(x, jnp.int32)
    i = jnp.int32(0x5F3759DF) - lax.shift_right_arithmetic(i, 1)
    y = plsc.bitcast(i, jnp.float32)
    y = y * (1.5 - 0.5 * x * y * y)
    y = y * (1.5 - 0.5 * x * y * y)
    y = y * (1.5 - 0.5 * x * y * y)
    return y


def _make_pass1():
    mesh = plsc.VectorSubcoreMesh(core_axis_name="c", subcore_axis_name="s")

    @functools.partial(
        pl.kernel,
        mesh=mesh,
        compiler_params=pltpu.CompilerParams(needs_layout_passes=False,
                                             use_tc_tiling_on_sc=False),
        out_type=[jax.ShapeDtypeStruct((_E,), jnp.float32),
                  jax.ShapeDtypeStruct((_E,), jnp.float32),
                  jax.ShapeDtypeStruct((2 * _NPAD,), jnp.float32),
                  jax.ShapeDtypeStruct((2 * _NPAD,), jnp.float32)],
        scratch_types=[
            pltpu.VMEM((_RSR_PAD,), jnp.float32),
            pltpu.VMEM((_RTG_PAD,), jnp.float32),
            [pltpu.VMEM((_C,), jnp.int32) for _ in range(2)],
            [pltpu.VMEM((_C,), jnp.float32) for _ in range(2)],
            [pltpu.VMEM((_C,), jnp.float32) for _ in range(2)],
            [pltpu.VMEM((_C,), jnp.float32) for _ in range(2)],
            [pltpu.VMEM((_IR, _IW), jnp.int32) for _ in range(2)],
            [pltpu.VMEM((_C,), jnp.float32) for _ in range(2)],
            pltpu.VMEM_SHARED((2 * _NPAD,), jnp.float32),
            [pltpu.SemaphoreType.DMA for _ in range(2)],
            [pltpu.SemaphoreType.DMA for _ in range(2)],
            [pltpu.SemaphoreType.DMA for _ in range(2)],
            [pltpu.SemaphoreType.DMA for _ in range(2)],
        ],
    )
    def pass1(wsr_hbm, wtg_hbm,
              rel_sr, conf_sr, imp_sr, pca_sr, rows_sr,
              rel_tg, conf_tg, imp_tg, pca_tg, rows_tg,
              zeros_hbm,
              vals_sr, vals_tg, deg_sr, deg_tg,
              wsr_v, wtg_v, rel_v, conf_v, imp_v, pca_v, rows_v, vals_v,
              deg_sh, dsem, rsem, osem, ssem):
        cid = lax.axis_index("c")
        sid = lax.axis_index("s")
        wid = sid * _NC + cid
        pltpu.sync_copy(zeros_hbm.at[pl.ds(sid * 2 * _SLICE, 2 * _SLICE)],
                        deg_sh.at[pl.ds(sid * 2 * _SLICE, 2 * _SLICE)])
        pltpu.sync_copy(wsr_hbm, wsr_v)
        pltpu.sync_copy(wtg_hbm, wtg_v)
        plsc.subcore_barrier()

        for w_v, rel_hbm, conf_hbm, imp_hbm, pca_hbm, rows2_hbm, vals_hbm, \
                doff in ((wsr_v, rel_sr, conf_sr, imp_sr, pca_sr, rows_sr,
                          vals_sr, 0),
                         (wtg_v, rel_tg, conf_tg, imp_tg, pca_tg, rows_tg,
                          vals_tg, _NPAD)):
            deg_side = deg_sh.at[pl.ds(doff, _NPAD)]

            def stage4(c, b):
                base = wid * _EPW + c * _C
                return (pltpu.make_async_copy(rel_hbm.at[pl.ds(base, _C)], rel_v[b], dsem[b]),
                        pltpu.make_async_copy(conf_hbm.at[pl.ds(base, _C)], conf_v[b], dsem[b]),
                        pltpu.make_async_copy(imp_hbm.at[pl.ds(base, _C)], imp_v[b], dsem[b]),
                        pltpu.make_async_copy(pca_hbm.at[pl.ds(base, _C)], pca_v[b], dsem[b]))

            def rows_cp(c, b):
                rbase = wid * _ERW + c * _IR
                return pltpu.make_async_copy(rows2_hbm.at[pl.ds(rbase, _IR)],
                                             rows_v[b], rsem[b])

            def wb_cp(c, b):
                base = wid * _EPW + c * _C
                return pltpu.make_async_copy(vals_v[b],
                                             vals_hbm.at[pl.ds(base, _C)],
                                             osem[b])

            def scat_cps(b):
                return [pltpu.make_async_copy(
                            vals_v[b].at[pl.ds(j * _IW, _IW)],
                            deg_side.at[rows_v[b].at[j]], ssem[b])
                        for j in range(_IR)]

            for b in range(2):
                for cp in stage4(b, b):
                    cp.start()
                rows_cp(b, b).start()

            @pl.loop(0, _NCHUNK, step=2)
            def _(g):
                for b in range(2):
                    c = g + b

                    @pl.when(c >= 2)
                    def _():
                        for cp in scat_cps(b):
                            cp.wait()
                        wb_cp(c - 2, b).wait()
                        rows_cp(c, b).start()

                    for cp in stage4(c, b):
                        cp.wait()

                    @plsc.parallel_loop(0, _C, _L, unroll=5)
                    def _(o):
                        att = plsc.load_gather(w_v, [rel_v[b][pl.ds(o, _L)]])
                        v = (conf_v[b][pl.ds(o, _L)] * imp_v[b][pl.ds(o, _L)]
                             * (0.5 * pca_v[b][pl.ds(o, _L)] + 0.5 * att))
                        vals_v[b][pl.ds(o, _L)] = v

                    rows_cp(c, b).wait()
                    wb_cp(c, b).start()
                    for cp in scat_cps(b):
                        cp.start(add=True)

                    @pl.when(c + 2 < _NCHUNK)
                    def _():
                        for cp in stage4(c + 2, b):
                            cp.start()

            for b in range(2):
                for cp in scat_cps(b):
                    cp.wait()
                wb_cp(_NCHUNK - 2 + b, b).wait()

        plsc.subcore_barrier()
        pltpu.sync_copy(deg_sh.at[pl.ds(sid * _SLICE, _SLICE)],
                        deg_sr.at[pl.ds(cid * _NPAD + sid * _SLICE, _SLICE)])
        pltpu.sync_copy(deg_sh.at[pl.ds(_NPAD + sid * _SLICE, _SLICE)],
                        deg_tg.at[pl.ds(cid * _NPAD + sid * _SLICE, _SLICE)])

    return pass1


def _make_pass2():
    mesh = plsc.VectorSubcoreMesh(core_axis_name="c", subcore_axis_name="s")

    @functools.partial(
        pl.kernel,
        mesh=mesh,
        compiler_params=pltpu.CompilerParams(needs_layout_passes=False,
                                             use_tc_tiling_on_sc=False),
        out_type=[jax.ShapeDtypeStruct((_E + _N,), jnp.float32),
                  jax.ShapeDtypeStruct((_E + _N,), jnp.float32)],
        scratch_types=[
            [pltpu.VMEM((_C,), jnp.int32) for _ in range(2)],
            [pltpu.VMEM((_C,), jnp.int32) for _ in range(2)],
            [pltpu.VMEM((_C,), jnp.float32) for _ in range(2)],
            [pltpu.VMEM((_C,), jnp.float32) for _ in range(2)],
            pltpu.VMEM((_NPAD,), jnp.float32),
            pltpu.VMEM((_ND,), jnp.float32),
            pltpu.VMEM_SHARED((_NPAD,), jnp.float32),
            [pltpu.SemaphoreType.DMA for _ in range(2)],
            [pltpu.SemaphoreType.DMA for _ in range(2)],
        ],
    )
    def pass2(deg_sr, rows_sr, cols_sr, vraw_sr,
              deg_tg, rows_tg, cols_tg, vraw_tg,
              vout_sr, vout_tg,
              rows_v, cols_v, vals_v, out_v, dis_full, diag_v, dis_sh,
              dsem, osem):
        cid = lax.axis_index("c")
        sid = lax.axis_index("s")
        wid = sid * _NC + cid
        _SUB = _SLICE // 4

        for deg_hbm, rows_hbm, cols_hbm, vraw_hbm, vout_hbm in (
                (deg_sr, rows_sr, cols_sr, vraw_sr, vout_sr),
                (deg_tg, rows_tg, cols_tg, vraw_tg, vout_tg)):
            for k in range(4):
                off = sid * _SLICE + k * _SUB
                pltpu.sync_copy(deg_hbm.at[pl.ds(off, _SUB)],
                                vals_v[0].at[pl.ds(0, _SUB)])
                pltpu.sync_copy(deg_hbm.at[pl.ds(_NPAD + off, _SUB)],
                                vals_v[1].at[pl.ds(0, _SUB)])

                @plsc.parallel_loop(0, _SUB, _L, unroll=7)
                def _(o):
                    d = (vals_v[0][pl.ds(o, _L)] + vals_v[1][pl.ds(o, _L)]
                         + 1.0)
                    y = _rsqrt_vec(jnp.maximum(d, 1e-12))
                    out_v[0][pl.ds(o, _L)] = jnp.where(d > 0.0, y, 0.0)

                pltpu.sync_copy(out_v[0].at[pl.ds(0, _SUB)],
                                dis_sh.at[pl.ds(off, _SUB)])
            plsc.subcore_barrier()
            pltpu.sync_copy(dis_sh, dis_full)
            plsc.subcore_barrier()

            @plsc.parallel_loop(0, _ND, _L, unroll=7)
            def _(o):
                y = dis_full[pl.ds(wid * _ND + o, _L)]
                diag_v[pl.ds(o, _L)] = y * y

            @pl.when(wid < _NW - 1)
            def _():
                pltpu.sync_copy(diag_v,
                                vout_hbm.at[pl.ds(_E + wid * _ND, _ND)])

            @pl.when(wid == _NW - 1)
            def _():
                pltpu.sync_copy(diag_v.at[pl.ds(0, _NLAST)],
                                vout_hbm.at[pl.ds(_E + (_NW - 1) * _ND,
                                                  _NLAST)])

            def stage3(c, b):
                base = wid * _EPW + c * _C
                return (pltpu.make_async_copy(rows_hbm.at[pl.ds(base, _C)], rows_v[b], dsem[b]),
                        pltpu.make_async_copy(cols_hbm.at[pl.ds(base, _C)], cols_v[b], dsem[b]),
                        pltpu.make_async_copy(vraw_hbm.at[pl.ds(base, _C)], vals_v[b], dsem[b]))

            def wb_cp(c, b):
                base = wid * _EPW + c * _C
                return pltpu.make_async_copy(out_v[b],
                                             vout_hbm.at[pl.ds(base, _C)],
                                             osem[b])

            for b in range(2):
                for cp in stage3(b, b):
                    cp.start()

            @pl.loop(0, _NCHUNK, step=2)
            def _(g):
                for b in range(2):
                    c = g + b

                    @pl.when(c >= 2)
                    def _():
                        wb_cp(c - 2, b).wait()

                    for cp in stage3(c, b):
                        cp.wait()

                    @plsc.parallel_loop(0, _C, _L, unroll=5)
                    def _(o):
                        dr = plsc.load_gather(dis_full, [rows_v[b][pl.ds(o, _L)]])
                        dc = plsc.load_gather(dis_full, [cols_v[b][pl.ds(o, _L)]])
                        out_v[b][pl.ds(o, _L)] = vals_v[b][pl.ds(o, _L)] * dr * dc

                    wb_cp(c, b).start()

                    @pl.when(c + 2 < _NCHUNK)
                    def _():
                        for cp in stage3(c + 2, b):
                            cp.start()

            for b in range(2):
                wb_cp(_NCHUNK - 2 + b, b).wait()

    return pass2


_pass1 = _make_pass1()
_pass2 = _make_pass2()


def kernel(rel_sr_weight, rel_tg_weight, pos_sr, relation_sr, conf_sr,
           imp_sr, pca_sr, pos_tg, relation_tg, conf_tg, imp_tg, pca_tg):
    w_sr, w_tg = _relation_w(rel_sr_weight, rel_tg_weight)
    zeros = jnp.zeros((2 * _NPAD,), jnp.float32)

    vraw_sr, vraw_tg, deg_sr, deg_tg = _pass1(
        w_sr, w_tg,
        relation_sr, conf_sr, imp_sr, pca_sr,
        pos_sr[0].reshape(_E // _IW, _IW),
        relation_tg, conf_tg, imp_tg, pca_tg,
        pos_tg[0].reshape(_E // _IW, _IW),
        zeros)
    diag = jnp.arange(_N, dtype=jnp.int32)
    vout_sr, vout_tg = _pass2(
        deg_sr, pos_sr[0], pos_sr[1], vraw_sr,
        deg_tg, pos_tg[0], pos_tg[1], vraw_tg)

    out = []
    for pos, vout in ((pos_sr, vout_sr), (pos_tg, vout_tg)):
        rows = jnp.concatenate([pos[0], diag])
        cols = jnp.concatenate([pos[1], diag])
        out.extend([rows, cols, vout])
    return tuple(out)
